# Initial kernel scaffold; baseline (speedup 1.0000x reference)
import jax
import jax.numpy as jnp
from jax.experimental import pallas as pl


def kernel(heatmap, offset, regression):
    raise NotImplementedError("write your pallas kernel here")


# speedup vs baseline: 5.5976x; 5.5976x over previous
import jax
import jax.numpy as jnp
from jax.experimental import pallas as pl
from jax.experimental.pallas import tpu as pltpu

_H = 128
_W = 128
_C = 80
_K = 100
_MINCONF = 0.3
_NBLK = 1280
_BLK = 1024
_NEG = -1.0
_BIG = 1 << 30


def _decode_kernel(hm_ref, off_ref, reg_ref,
                   scores_ref, classes_ref, xmin_ref, ymin_ref,
                   w_ref, h_ref, v_ref):
    a = hm_ref[...]
    ninf = jnp.float32(-jnp.inf)

    left = jnp.concatenate([jnp.full((_H, _C), ninf, jnp.float32),
                            a[:, :-_C]], axis=1)
    right = jnp.concatenate([a[:, _C:],
                             jnp.full((_H, _C), ninf, jnp.float32)], axis=1)
    cm = jnp.maximum(a, jnp.maximum(left, right))
    up = jnp.concatenate([jnp.full((1, _W * _C), ninf, jnp.float32),
                          cm[:-1, :]], axis=0)
    down = jnp.concatenate([cm[1:, :],
                            jnp.full((1, _W * _C), ninf, jnp.float32)], axis=0)
    hmax = jnp.maximum(cm, jnp.maximum(up, down))
    v = jnp.where(a == hmax, a, jnp.float32(0.0))

    v_ref[...] = v.reshape(_NBLK, 8, 128)
    bmax = jnp.max(v.reshape(_H, 10, _BLK), axis=2)
    l1_0 = bmax.reshape(8, 160)

    iota_b = jax.lax.broadcasted_iota(jnp.int32, (8, 160), 0) * 160 + \
        jax.lax.broadcasted_iota(jnp.int32, (8, 160), 1)
    iota_o = jax.lax.broadcasted_iota(jnp.int32, (8, 128), 0) * 128 + \
        jax.lax.broadcasted_iota(jnp.int32, (8, 128), 1)
    lane = jax.lax.broadcasted_iota(jnp.int32, (1, 2 * _W), 1)

    zvec = jnp.zeros((1, 128), jnp.float32)
    carry0 = (l1_0, zvec, jnp.zeros((1, 128), jnp.int32),
              zvec, zvec, zvec, zvec)
    out_lane = jax.lax.broadcasted_iota(jnp.int32, (1, 128), 1)

    def body(i, carry):
        l1v, s_v, c_v, x0_v, y0_v, w_v, h_v = carry
        g = jnp.max(l1v)
        b = jnp.min(jnp.where(l1v == g, iota_b, _BIG))
        blk = v_ref[pl.ds(b, 1)].reshape(8, 128)
        o = jnp.min(jnp.where(blk == g, iota_o, _BIG))
        flat = b * _BLK + o

        newblk = jnp.where(iota_o == o, _NEG, blk)
        v_ref[pl.ds(b, 1)] = newblk.reshape(1, 8, 128)
        nb = jnp.max(newblk)
        l1v = jnp.where(iota_b == b, nb, l1v)

        c = flat % _C
        x = (flat // _C) % _W
        y = flat // (_W * _C)
        offrow = off_ref[pl.ds(x, 1)].reshape(1, 2 * _W)
        regrow = reg_ref[pl.ds(x, 1)].reshape(1, 2 * _W)
        zf = jnp.float32(0.0)
        ox = jnp.sum(jnp.where(lane == 2 * y, offrow, zf))
        oy = jnp.sum(jnp.where(lane == 2 * y + 1, offrow, zf))
        rx = jnp.sum(jnp.where(lane == 2 * y, regrow, zf))
        ry = jnp.sum(jnp.where(lane == 2 * y + 1, regrow, zf))
        keep = g >= _MINCONF
        xf = x.astype(jnp.float32)
        yf = y.astype(jnp.float32)
        xmin = jnp.where(keep, xf + ox - rx * 0.5, zf)
        ymin = jnp.where(keep, yf + oy - ry * 0.5, zf)
        ww = jnp.where(keep, rx, zf)
        hh = jnp.where(keep, ry, zf)
        here = out_lane == i
        return (l1v,
                jnp.where(here, jnp.where(keep, g, zf), s_v),
                jnp.where(here, jnp.where(keep, c, 0), c_v),
                jnp.where(here, xmin, x0_v),
                jnp.where(here, ymin, y0_v),
                jnp.where(here, ww, w_v),
                jnp.where(here, hh, h_v))

    out = jax.lax.fori_loop(0, _K, body, carry0)
    scores_ref[...] = out[1]
    classes_ref[...] = out[2]
    xmin_ref[...] = out[3]
    ymin_ref[...] = out[4]
    w_ref[...] = out[5]
    h_ref[...] = out[6]


@jax.jit
def kernel(heatmap, offset, regression):
    hm2 = heatmap.reshape(_H, _W * _C)
    off3 = offset.reshape(_H, 1, _W * 2)
    reg3 = regression.reshape(_H, 1, _W * 2)
    vec = jax.ShapeDtypeStruct((1, 128), jnp.float32)
    scores_p, classes_p, xmin_p, ymin_p, w_p, h_p = pl.pallas_call(
        _decode_kernel,
        out_shape=(
            vec,
            jax.ShapeDtypeStruct((1, 128), jnp.int32),
            vec, vec, vec, vec,
        ),
        scratch_shapes=[
            pltpu.VMEM((_NBLK, 8, 128), jnp.float32),
        ],
    )(hm2, off3, reg3)
    scores = scores_p[:, :_K]
    classes = classes_p[:, :_K]
    bboxes = jnp.stack([xmin_p[0, :_K], ymin_p[0, :_K],
                        w_p[0, :_K], h_p[0, :_K]], axis=-1)[None]
    return bboxes, scores, classes


# speedup vs baseline: 6.3631x; 1.1368x over previous
import jax
import jax.numpy as jnp
from jax.experimental import pallas as pl
from jax.experimental.pallas import tpu as pltpu

_H = 128
_W = 128
_C = 80
_K = 100
_MINCONF = 0.3
_NBLK = 1280
_BLK = 1024
_NEG = -1.0
_BIG = 1 << 30


def _decode_kernel(hm_ref, off_ref, reg_ref, out_ref, v_ref):
    a = hm_ref[...]
    ninf = jnp.float32(-jnp.inf)

    left = jnp.concatenate([jnp.full((_H, _C), ninf, jnp.float32),
                            a[:, :-_C]], axis=1)
    right = jnp.concatenate([a[:, _C:],
                             jnp.full((_H, _C), ninf, jnp.float32)], axis=1)
    cm = jnp.maximum(a, jnp.maximum(left, right))
    up = jnp.concatenate([jnp.full((1, _W * _C), ninf, jnp.float32),
                          cm[:-1, :]], axis=0)
    down = jnp.concatenate([cm[1:, :],
                            jnp.full((1, _W * _C), ninf, jnp.float32)], axis=0)
    hmax = jnp.maximum(cm, jnp.maximum(up, down))
    v = jnp.where(a == hmax, a, jnp.float32(0.0))

    v_ref[...] = v.reshape(_NBLK, 8, 128)
    l1_0 = jnp.max(v.reshape(8, 160, 8, 128), axis=(2, 3))

    iota_b = jax.lax.broadcasted_iota(jnp.int32, (8, 160), 0) * 160 + \
        jax.lax.broadcasted_iota(jnp.int32, (8, 160), 1)
    iota_o = jax.lax.broadcasted_iota(jnp.int32, (8, 128), 0) * 128 + \
        jax.lax.broadcasted_iota(jnp.int32, (8, 128), 1)
    row_col = jax.lax.broadcasted_iota(jnp.int32, (128, 1), 0)

    carry0 = (l1_0, jnp.zeros((128, 1), jnp.float32),
              jnp.zeros((128, 1), jnp.int32))

    def body(i, carry):
        l1v, s_col, f_col = carry
        g = jnp.max(l1v)
        b = jnp.min(jnp.where(l1v == g, iota_b, _BIG))
        blk = v_ref[pl.ds(b, 1)].reshape(8, 128)
        eq = blk == g
        o = jnp.min(jnp.where(eq, iota_o, _BIG))
        v_ref[pl.ds(b, 1)] = jnp.where(
            iota_o == o, _NEG, blk).reshape(1, 8, 128)
        cnt = jnp.sum(jnp.where(eq, 1, 0))
        second = jnp.max(jnp.where(eq, _NEG, blk))
        nb = jnp.where(cnt > 1, g, second)
        l1v = jnp.where(iota_b == b, nb, l1v)
        here = row_col == i
        return (l1v,
                jnp.where(here, g, s_col),
                jnp.where(here, b * _BLK + o, f_col))

    l1v, s_col, f_col = jax.lax.fori_loop(0, _K, body, carry0)

    c = f_col % _C
    x = (f_col // _C) % _W
    y = f_col // (_W * _C)
    lane128 = jax.lax.broadcasted_iota(jnp.int32, (128, 128), 1)
    sel = (lane128 == x).astype(jnp.float32)
    rows_off = jax.lax.dot(sel, off_ref[...],
                           preferred_element_type=jnp.float32)
    rows_reg = jax.lax.dot(sel, reg_ref[...],
                           preferred_element_type=jnp.float32)
    lane256 = jax.lax.broadcasted_iota(jnp.int32, (128, 256), 1)
    zf = jnp.float32(0.0)
    mx = lane256 == 2 * y
    my = lane256 == 2 * y + 1
    ox = jnp.sum(jnp.where(mx, rows_off, zf), axis=1, keepdims=True)
    oy = jnp.sum(jnp.where(my, rows_off, zf), axis=1, keepdims=True)
    rx = jnp.sum(jnp.where(mx, rows_reg, zf), axis=1, keepdims=True)
    ry = jnp.sum(jnp.where(my, rows_reg, zf), axis=1, keepdims=True)
    keep = s_col >= _MINCONF
    xmin = jnp.where(keep, x.astype(jnp.float32) + ox - rx * 0.5, zf)
    ymin = jnp.where(keep, y.astype(jnp.float32) + oy - ry * 0.5, zf)
    ww = jnp.where(keep, rx, zf)
    hh = jnp.where(keep, ry, zf)
    sc = jnp.where(keep, s_col, zf)
    cf = jnp.where(keep, c, 0).astype(jnp.float32)
    out_ref[...] = jnp.concatenate(
        [xmin, ymin, ww, hh, sc, cf, jnp.zeros((128, 2), jnp.float32)],
        axis=1)


@jax.jit
def kernel(heatmap, offset, regression):
    hm2 = heatmap.reshape(_H, _W * _C)
    off2 = offset.reshape(_H, _W * 2)
    reg2 = regression.reshape(_H, _W * 2)
    out = pl.pallas_call(
        _decode_kernel,
        out_shape=jax.ShapeDtypeStruct((128, 8), jnp.float32),
        scratch_shapes=[
            pltpu.VMEM((_NBLK, 8, 128), jnp.float32),
        ],
    )(hm2, off2, reg2)
    bboxes = out[:_K, 0:4][None]
    scores = out[:_K, 4][None]
    classes = out[:_K, 5].astype(jnp.int32)[None]
    return bboxes, scores, classes
